# baseline (speedup 1.0000x reference)
import functools

import jax
import jax.numpy as jnp
from jax.experimental import pallas as pl
from jax.experimental.pallas import tpu as pltpu

N_NODES = 10000
EMB = 128


def _mm_body(x_ref, w_ref, o_ref):
    o_ref[...] = jax.lax.dot_general(
        x_ref[...], w_ref[...],
        dimension_numbers=(((1,), (1,)), ((), ())),
        preferred_element_type=jnp.float32,
    )


def _linear(mmsbm_val, W):
    M = mmsbm_val.shape[0]
    BM = 640
    assert M % BM == 0
    grid = (M // BM,)
    return pl.pallas_call(
        _mm_body,
        grid=grid,
        in_specs=[
            pl.BlockSpec((BM, EMB), lambda i: (i, 0)),
            pl.BlockSpec((128, EMB), lambda i: (0, 0)),
        ],
        out_specs=pl.BlockSpec((BM, EMB), lambda i: (i, 0)),
        out_shape=jax.ShapeDtypeStruct((M, EMB), jnp.float32),
    )(mmsbm_val, W)


def kernel(mmsbm_index, mmsbm_val, edge_index, edge_attr, W, num_nodes):
    N = N_NODES
    mv = _linear(mmsbm_val, W)
    ar = jnp.arange(N, dtype=edge_index.dtype)
    loop_idx = jnp.stack([ar, ar])
    loop_attr = jnp.zeros((N, edge_attr.shape[1]), dtype=edge_attr.dtype)
    ei = jnp.concatenate([edge_index, loop_idx], axis=1)
    ea = jnp.concatenate([edge_attr, loop_attr], axis=0)
    all_idx = jnp.concatenate([ei, mmsbm_index], axis=1)
    all_val = jnp.concatenate([ea, mv], axis=0)
    keys = all_idx[0] * num_nodes + all_idx[1]
    total = keys.shape[0]
    uk, inv = jnp.unique(keys, return_inverse=True, size=total, fill_value=N * N)
    out_val = jax.ops.segment_sum(all_val, inv.reshape(-1), num_segments=total)
    out_idx = jnp.stack([uk // num_nodes, uk % num_nodes])
    return out_idx, out_val
